# Initial kernel scaffold; baseline (speedup 1.0000x reference)
import functools

import jax
import jax.numpy as jnp
from jax import lax
from jax.experimental import pallas as pl
from jax.experimental.pallas import tpu as pltpu
from jax.experimental.pallas import tpu_sc as plsc

N = 10000
D = 128
E = 320000

NC = 2
NS = 16
NW = NC * NS
CH = 128
CPW = 80
EPW = CH * CPW
EPAD = NW * EPW
NP = 10240
RPT = NP // NS
ZR = 64

_mesh = plsc.VectorSubcoreMesh(
    core_axis_name="c", subcore_axis_name="s", num_cores=NC, num_subcores=NS)


def _zero_rows16(ref, nrows):
    zv = jnp.zeros((16,), jnp.float32)

    def body(r, carry):
        ref[r] = zv
        return carry

    lax.fori_loop(0, nrows, body, 0)


def _make_sc_segsum(with_cnt):
    out_type = [jax.ShapeDtypeStruct((NC, NP, D), jnp.float32)]
    scratch = [
        pltpu.VMEM((CPW, CH), jnp.int32),
        pltpu.VMEM((CPW, CH), jnp.int32),
        pltpu.VMEM((2, CH, D), jnp.float32),
        pltpu.VMEM((ZR, D), jnp.float32),
        pltpu.VMEM_SHARED((NP, D), jnp.float32),
        pltpu.SemaphoreType.DMA,
        pltpu.SemaphoreType.DMA,
    ]
    if with_cnt:
        out_type.append(jax.ShapeDtypeStruct((NC, NP, 16), jnp.float32))
        scratch += [
            pltpu.VMEM((RPT, 16), jnp.float32),
            pltpu.VMEM((CH, 16), jnp.float32),
            pltpu.VMEM_SHARED((NP, 16), jnp.float32),
        ]

    @functools.partial(
        pl.kernel, out_type=tuple(out_type), mesh=_mesh,
        scratch_types=tuple(scratch))
    def sc_segsum(x_hbm, srcs_hbm, dsts_hbm, *rest):
        if with_cnt:
            (out_hbm, cnt_hbm, src_v, dst_v, rows_v, zbuf, acc_sh,
             sem0, sem1, zcnt, ones_v, cnt_sh) = rest
        else:
            (out_hbm, src_v, dst_v, rows_v, zbuf, acc_sh, sem0, sem1) = rest

        core = lax.axis_index("c")
        sub = lax.axis_index("s")
        wid = sub * NC + core

        zv = jnp.zeros((16,), jnp.float32)

        def zrow(r, carry):
            def zcol(cc, carry2):
                zbuf[r, pl.ds(cc * 16, 16)] = zv
                return carry2
            return lax.fori_loop(0, D // 16, zcol, carry)

        lax.fori_loop(0, ZR, zrow, 0)
        if with_cnt:
            _zero_rows16(zcnt, RPT)
            ov = jnp.ones((16,), jnp.float32)

            def orow(r, carry):
                ones_v[r] = ov
                return carry

            lax.fori_loop(0, CH, orow, 0)

        pltpu.sync_copy(srcs_hbm.at[wid], src_v)
        pltpu.sync_copy(dsts_hbm.at[wid], dst_v)

        base = sub * RPT
        for k in range(RPT // ZR):
            pltpu.sync_copy(zbuf, acc_sh.at[pl.ds(base + k * ZR, ZR)])
        if with_cnt:
            pltpu.sync_copy(zcnt, cnt_sh.at[pl.ds(base, RPT)])
        plsc.subcore_barrier()

        sems = (sem0, sem1)
        pltpu.async_copy(x_hbm.at[src_v.at[0]], rows_v.at[0], sem0)

        def chunk(ci, buf):
            pltpu.make_async_copy(
                x_hbm.at[src_v.at[ci]], rows_v.at[buf], sems[buf]).wait()

            @pl.when(ci + 1 < CPW)
            def _():
                pltpu.async_copy(
                    x_hbm.at[src_v.at[ci + 1]], rows_v.at[1 - buf],
                    sems[1 - buf])

            pltpu.sync_copy(rows_v.at[buf], acc_sh.at[dst_v.at[ci]], add=True)
            if with_cnt:
                pltpu.sync_copy(ones_v, cnt_sh.at[dst_v.at[ci]], add=True)

        def body(g, carry):
            chunk(2 * g, 0)
            chunk(2 * g + 1, 1)
            return carry

        lax.fori_loop(0, CPW // 2, body, 0)
        plsc.subcore_barrier()

        pltpu.sync_copy(acc_sh.at[pl.ds(base, RPT)],
                        out_hbm.at[core, pl.ds(base, RPT)])
        if with_cnt:
            pltpu.sync_copy(cnt_sh.at[pl.ds(base, RPT)],
                            cnt_hbm.at[core, pl.ds(base, RPT)])

    return sc_segsum


_sc_segsum_cnt = _make_sc_segsum(True)
_sc_segsum = _make_sc_segsum(False)

_R = 400


def _dense_ln_body(p_ref, cnt_ref, x_ref, wl_ref, wr_ref, bl_ref, g_ref,
                   b_ref, o_ref):
    s = p_ref[0] + p_ref[1]
    c = cnt_ref[0, :, 0:1] + cnt_ref[1, :, 0:1]
    mean = s / jnp.maximum(c, 1.0)
    x = x_ref[...]
    h = jnp.dot(mean, wl_ref[...], preferred_element_type=jnp.float32)
    h = h + jnp.dot(x, wr_ref[...], preferred_element_type=jnp.float32)
    h = h + bl_ref[...]
    m = jnp.mean(h, axis=-1, keepdims=True)
    v = jnp.mean((h - m) * (h - m), axis=-1, keepdims=True)
    hn = (h - m) / jnp.sqrt(v + 1e-5) * g_ref[...] + b_ref[...]
    o_ref[...] = jnp.maximum(hn, 0.0) + x


def _dense_plain_body(p_ref, cnt_ref, x_ref, wl_ref, wr_ref, bl_ref, o_ref):
    s = p_ref[0] + p_ref[1]
    c = cnt_ref[0, :, 0:1] + cnt_ref[1, :, 0:1]
    mean = s / jnp.maximum(c, 1.0)
    h = jnp.dot(mean, wl_ref[...], preferred_element_type=jnp.float32)
    h = h + jnp.dot(x_ref[...], wr_ref[...], preferred_element_type=jnp.float32)
    o_ref[...] = h + bl_ref[...]


def _dense(body, n_extra):
    in_specs = [
        pl.BlockSpec((NC, _R, D), lambda i: (0, i, 0)),
        pl.BlockSpec((NC, _R, 16), lambda i: (0, i, 0)),
        pl.BlockSpec((_R, D), lambda i: (i, 0)),
        pl.BlockSpec((D, D), lambda i: (0, 0)),
        pl.BlockSpec((D, D), lambda i: (0, 0)),
        pl.BlockSpec((1, D), lambda i: (0, 0)),
    ]
    in_specs += [pl.BlockSpec((1, D), lambda i: (0, 0))] * n_extra
    return pl.pallas_call(
        body,
        grid=(N // _R,),
        in_specs=in_specs,
        out_specs=pl.BlockSpec((_R, D), lambda i: (i, 0)),
        out_shape=jax.ShapeDtypeStruct((N, D), jnp.float32),
    )


_dense_ln = _dense(_dense_ln_body, 2)
_dense_plain = _dense(_dense_plain_body, 0)


def kernel(x, edge_index, Wl0, bl0, Wr0, Wl1, bl1, Wr1, Wl2, bl2, Wr2,
           g0, b0, g1, b1):
    src = edge_index[0]
    dst = edge_index[1]
    pad = EPAD - E
    srcp = jnp.concatenate(
        [src, jnp.zeros((pad,), jnp.int32)]).reshape(NW, CPW, CH)
    dstp = jnp.concatenate(
        [dst, jnp.full((pad,), N, jnp.int32)]).reshape(NW, CPW, CH)

    def r(a):
        return a.reshape(1, D)

    p, cnt = _sc_segsum_cnt(x, srcp, dstp)
    x1 = _dense_ln(p, cnt, x, Wl0.T, Wr0.T, r(bl0), r(g0), r(b0))
    (p,) = _sc_segsum(x1, srcp, dstp)
    x2 = _dense_ln(p, cnt, x1, Wl1.T, Wr1.T, r(bl1), r(g1), r(b1))
    (p,) = _sc_segsum(x2, srcp, dstp)
    return _dense_plain(p, cnt, x2, Wl2.T, Wr2.T, r(bl2))


# speedup vs baseline: 3.3135x; 3.3135x over previous
import functools

import jax
import jax.numpy as jnp
from jax import lax
from jax.experimental import pallas as pl
from jax.experimental.pallas import tpu as pltpu
from jax.experimental.pallas import tpu_sc as plsc

N = 10000
D = 128
E = 320000

NC = 2
NS = 16
NW = NC * NS
CH = 64
CPW = 160
EPW = CH * CPW
EPAD = NW * EPW
NP = 10112
RPT = NP // NS
ZR = 64

_mesh = plsc.VectorSubcoreMesh(
    core_axis_name="c", subcore_axis_name="s", num_cores=NC, num_subcores=NS)


def _zero_rows16(ref, nrows):
    zv = jnp.zeros((16,), jnp.float32)

    def body(r, carry):
        ref[r] = zv
        return carry

    lax.fori_loop(0, nrows, body, 0)


def _make_sc_segsum(with_cnt):
    out_type = [jax.ShapeDtypeStruct((NC, NP, D), jnp.float32)]
    scratch = [
        pltpu.VMEM((CPW, CH), jnp.int32),
        pltpu.VMEM((CPW, CH), jnp.int32),
        pltpu.VMEM((2, CH, D), jnp.float32),
        pltpu.VMEM_SHARED((NP, D), jnp.float32),
        pltpu.SemaphoreType.DMA,
        pltpu.SemaphoreType.DMA,
    ]
    if with_cnt:
        out_type.append(jax.ShapeDtypeStruct((NC, NP, 16), jnp.float32))
        scratch += [
            pltpu.VMEM((ZR, 16), jnp.float32),
            pltpu.VMEM((CH, 16), jnp.float32),
            pltpu.VMEM_SHARED((NP, 16), jnp.float32),
        ]

    @functools.partial(
        pl.kernel, out_type=tuple(out_type), mesh=_mesh,
        scratch_types=tuple(scratch),
        compiler_params=pltpu.CompilerParams(use_tc_tiling_on_sc=False))
    def sc_segsum(x_hbm, srcs_hbm, dsts_hbm, *rest):
        if with_cnt:
            (out_hbm, cnt_hbm, src_v, dst_v, rows_v, acc_sh,
             sem0, sem1, zcnt, ones_v, cnt_sh) = rest
        else:
            (out_hbm, src_v, dst_v, rows_v, acc_sh, sem0, sem1) = rest

        core = lax.axis_index("c")
        sub = lax.axis_index("s")
        wid = sub * NC + core

        zv = jnp.zeros((16,), jnp.float32)

        def zrow(r, carry):
            def zcol(cc, carry2):
                rows_v[0, r, pl.ds(cc * 16, 16)] = zv
                return carry2
            return lax.fori_loop(0, D // 16, zcol, carry)

        lax.fori_loop(0, ZR, zrow, 0)
        if with_cnt:
            _zero_rows16(zcnt, ZR)
            ov = jnp.ones((16,), jnp.float32)

            def orow(r, carry):
                ones_v[r] = ov
                return carry

            lax.fori_loop(0, CH, orow, 0)

        pltpu.sync_copy(srcs_hbm.at[wid], src_v)
        pltpu.sync_copy(dsts_hbm.at[wid], dst_v)

        base = sub * RPT
        offs = [k * ZR for k in range(RPT // ZR)]
        if RPT % ZR:
            offs.append(RPT - ZR)
        for off in offs:
            pltpu.sync_copy(rows_v.at[0], acc_sh.at[pl.ds(base + off, ZR)])
            if with_cnt:
                pltpu.sync_copy(zcnt, cnt_sh.at[pl.ds(base + off, ZR)])
        plsc.subcore_barrier()

        sems = (sem0, sem1)
        pltpu.async_copy(x_hbm.at[src_v.at[0]], rows_v.at[0], sem0)

        def chunk(ci, buf):
            pltpu.make_async_copy(
                x_hbm.at[src_v.at[ci]], rows_v.at[buf], sems[buf]).wait()

            @pl.when(ci + 1 < CPW)
            def _():
                pltpu.async_copy(
                    x_hbm.at[src_v.at[ci + 1]], rows_v.at[1 - buf],
                    sems[1 - buf])

            pltpu.sync_copy(rows_v.at[buf], acc_sh.at[dst_v.at[ci]], add=True)
            if with_cnt:
                pltpu.sync_copy(ones_v, cnt_sh.at[dst_v.at[ci]], add=True)

        def body(g, carry):
            chunk(2 * g, 0)
            chunk(2 * g + 1, 1)
            return carry

        lax.fori_loop(0, CPW // 2, body, 0)
        plsc.subcore_barrier()

        pltpu.sync_copy(acc_sh.at[pl.ds(base, RPT)],
                        out_hbm.at[core, pl.ds(base, RPT)])
        if with_cnt:
            pltpu.sync_copy(cnt_sh.at[pl.ds(base, RPT)],
                            cnt_hbm.at[core, pl.ds(base, RPT)])

    return sc_segsum


_sc_segsum_cnt = _make_sc_segsum(True)
_sc_segsum = _make_sc_segsum(False)

_R = 400


def _dense_ln_body(p_ref, cnt_ref, x_ref, wl_ref, wr_ref, bl_ref, g_ref,
                   b_ref, o_ref):
    s = p_ref[0] + p_ref[1]
    c = cnt_ref[0, :, 0:1] + cnt_ref[1, :, 0:1]
    mean = s / jnp.maximum(c, 1.0)
    x = x_ref[...]
    h = jnp.dot(mean, wl_ref[...], preferred_element_type=jnp.float32)
    h = h + jnp.dot(x, wr_ref[...], preferred_element_type=jnp.float32)
    h = h + bl_ref[...]
    m = jnp.mean(h, axis=-1, keepdims=True)
    v = jnp.mean((h - m) * (h - m), axis=-1, keepdims=True)
    hn = (h - m) / jnp.sqrt(v + 1e-5) * g_ref[...] + b_ref[...]
    o_ref[...] = jnp.maximum(hn, 0.0) + x


def _dense_plain_body(p_ref, cnt_ref, x_ref, wl_ref, wr_ref, bl_ref, o_ref):
    s = p_ref[0] + p_ref[1]
    c = cnt_ref[0, :, 0:1] + cnt_ref[1, :, 0:1]
    mean = s / jnp.maximum(c, 1.0)
    h = jnp.dot(mean, wl_ref[...], preferred_element_type=jnp.float32)
    h = h + jnp.dot(x_ref[...], wr_ref[...], preferred_element_type=jnp.float32)
    o_ref[...] = h + bl_ref[...]


def _dense(body, n_extra):
    in_specs = [
        pl.BlockSpec((NC, _R, D), lambda i: (0, i, 0)),
        pl.BlockSpec((NC, _R, 16), lambda i: (0, i, 0)),
        pl.BlockSpec((_R, D), lambda i: (i, 0)),
        pl.BlockSpec((D, D), lambda i: (0, 0)),
        pl.BlockSpec((D, D), lambda i: (0, 0)),
        pl.BlockSpec((1, D), lambda i: (0, 0)),
    ]
    in_specs += [pl.BlockSpec((1, D), lambda i: (0, 0))] * n_extra
    return pl.pallas_call(
        body,
        grid=(N // _R,),
        in_specs=in_specs,
        out_specs=pl.BlockSpec((_R, D), lambda i: (i, 0)),
        out_shape=jax.ShapeDtypeStruct((N, D), jnp.float32),
    )


_dense_ln = _dense(_dense_ln_body, 2)
_dense_plain = _dense(_dense_plain_body, 0)


def kernel(x, edge_index, Wl0, bl0, Wr0, Wl1, bl1, Wr1, Wl2, bl2, Wr2,
           g0, b0, g1, b1):
    src = edge_index[0]
    dst = edge_index[1]
    pad = EPAD - E
    srcp = jnp.concatenate(
        [src, jnp.zeros((pad,), jnp.int32)]).reshape(NW, CPW, CH)
    dstp = jnp.concatenate(
        [dst, jnp.full((pad,), N, jnp.int32)]).reshape(NW, CPW, CH)

    def r(a):
        return a.reshape(1, D)

    p, cnt = _sc_segsum_cnt(x, srcp, dstp)
    x1 = _dense_ln(p, cnt, x, Wl0.T, Wr0.T, r(bl0), r(g0), r(b0))
    (p,) = _sc_segsum(x1, srcp, dstp)
    x2 = _dense_ln(p, cnt, x1, Wl1.T, Wr1.T, r(bl1), r(g1), r(b1))
    (p,) = _sc_segsum(x2, srcp, dstp)
    return _dense_plain(p, cnt, x2, Wl2.T, Wr2.T, r(bl2))


# speedup vs baseline: 5.0021x; 1.5096x over previous
import functools

import jax
import jax.numpy as jnp
from jax import lax
from jax.experimental import pallas as pl
from jax.experimental.pallas import tpu as pltpu
from jax.experimental.pallas import tpu_sc as plsc

N = 10000
D = 128
E = 320000

NC = 2
NS = 16
NW = NC * NS
CH = 64
CPW = 158
EPW = CH * CPW
EPAD = NW * EPW
NP = 10112
RPT = NP // NS
ZR = 64

_mesh = plsc.VectorSubcoreMesh(
    core_axis_name="c", subcore_axis_name="s", num_cores=NC, num_subcores=NS)


def _zero_rows16(ref, nrows):
    zv = jnp.zeros((16,), jnp.float32)

    def body(r, carry):
        ref[r] = zv
        return carry

    lax.fori_loop(0, nrows, body, 0)


def _make_sc_segsum(with_cnt):
    out_type = [jax.ShapeDtypeStruct((NC, NP, D), jnp.float32)]
    scratch = [
        pltpu.VMEM((CPW, CH), jnp.int32),
        pltpu.VMEM((CPW, CH), jnp.int32),
        pltpu.VMEM((2, CH, D), jnp.float32),
        pltpu.VMEM_SHARED((NP, D), jnp.float32),
        pltpu.SemaphoreType.DMA,
        pltpu.SemaphoreType.DMA,
    ]
    if with_cnt:
        out_type.append(jax.ShapeDtypeStruct((NC, NP, 16), jnp.float32))
        scratch += [
            pltpu.VMEM((ZR, 16), jnp.float32),
            pltpu.VMEM((CH, 16), jnp.float32),
            pltpu.VMEM_SHARED((NP, 16), jnp.float32),
        ]

    @functools.partial(
        pl.kernel, out_type=tuple(out_type), mesh=_mesh,
        scratch_types=tuple(scratch),
        compiler_params=pltpu.CompilerParams(use_tc_tiling_on_sc=False))
    def sc_segsum(x_hbm, srcs_hbm, dsts_hbm, *rest):
        if with_cnt:
            (out_hbm, cnt_hbm, src_v, dst_v, rows_v, acc_sh,
             sem0, sem1, zcnt, ones_v, cnt_sh) = rest
        else:
            (out_hbm, src_v, dst_v, rows_v, acc_sh, sem0, sem1) = rest

        core = lax.axis_index("c")
        sub = lax.axis_index("s")
        wid = sub * NC + core

        zv = jnp.zeros((16,), jnp.float32)

        def zrow(r, carry):
            def zcol(cc, carry2):
                rows_v[0, r, pl.ds(cc * 16, 16)] = zv
                return carry2
            return lax.fori_loop(0, D // 16, zcol, carry)

        lax.fori_loop(0, ZR, zrow, 0)
        if with_cnt:
            _zero_rows16(zcnt, ZR)
            ov = jnp.ones((16,), jnp.float32)

            def orow(r, carry):
                ones_v[r] = ov
                return carry

            lax.fori_loop(0, CH, orow, 0)

        pltpu.sync_copy(srcs_hbm.at[wid], src_v)
        pltpu.sync_copy(dsts_hbm.at[wid], dst_v)

        base = sub * RPT
        offs = [k * ZR for k in range(RPT // ZR)]
        if RPT % ZR:
            offs.append(RPT - ZR)
        for off in offs:
            pltpu.sync_copy(rows_v.at[0], acc_sh.at[pl.ds(base + off, ZR)])
            if with_cnt:
                pltpu.sync_copy(zcnt, cnt_sh.at[pl.ds(base + off, ZR)])
        plsc.subcore_barrier()

        sems = (sem0, sem1)
        pltpu.async_copy(x_hbm.at[src_v.at[0]], rows_v.at[0], sem0)

        def chunk(ci, buf):
            pltpu.make_async_copy(
                x_hbm.at[src_v.at[ci]], rows_v.at[buf], sems[buf]).wait()

            @pl.when(ci + 1 < CPW)
            def _():
                pltpu.async_copy(
                    x_hbm.at[src_v.at[ci + 1]], rows_v.at[1 - buf],
                    sems[1 - buf])

            pltpu.sync_copy(rows_v.at[buf], acc_sh.at[dst_v.at[ci]], add=True)
            if with_cnt:
                pltpu.sync_copy(ones_v, cnt_sh.at[dst_v.at[ci]], add=True)

        def body(g, carry):
            chunk(2 * g, 0)
            chunk(2 * g + 1, 1)
            return carry

        lax.fori_loop(0, CPW // 2, body, 0)
        plsc.subcore_barrier()

        pltpu.sync_copy(acc_sh.at[pl.ds(base, RPT)],
                        out_hbm.at[core, pl.ds(base, RPT)])
        if with_cnt:
            pltpu.sync_copy(cnt_sh.at[pl.ds(base, RPT)],
                            cnt_hbm.at[core, pl.ds(base, RPT)])

    return sc_segsum


_sc_segsum_cnt = _make_sc_segsum(True)
_sc_segsum = _make_sc_segsum(False)

_R = 400


def _dense_ln_body(p_ref, cnt_ref, x_ref, wl_ref, wr_ref, bl_ref, g_ref,
                   b_ref, o_ref):
    s = p_ref[0] + p_ref[1]
    c = cnt_ref[0, :, 0:1] + cnt_ref[1, :, 0:1]
    mean = s / jnp.maximum(c, 1.0)
    x = x_ref[...]
    h = jnp.dot(mean, wl_ref[...], preferred_element_type=jnp.float32)
    h = h + jnp.dot(x, wr_ref[...], preferred_element_type=jnp.float32)
    h = h + bl_ref[...]
    m = jnp.mean(h, axis=-1, keepdims=True)
    v = jnp.mean((h - m) * (h - m), axis=-1, keepdims=True)
    hn = (h - m) / jnp.sqrt(v + 1e-5) * g_ref[...] + b_ref[...]
    o_ref[...] = jnp.maximum(hn, 0.0) + x


def _dense_plain_body(p_ref, cnt_ref, x_ref, wl_ref, wr_ref, bl_ref, o_ref):
    s = p_ref[0] + p_ref[1]
    c = cnt_ref[0, :, 0:1] + cnt_ref[1, :, 0:1]
    mean = s / jnp.maximum(c, 1.0)
    h = jnp.dot(mean, wl_ref[...], preferred_element_type=jnp.float32)
    h = h + jnp.dot(x_ref[...], wr_ref[...], preferred_element_type=jnp.float32)
    o_ref[...] = h + bl_ref[...]


def _dense(body, n_extra):
    in_specs = [
        pl.BlockSpec((NC, _R, D), lambda i: (0, i, 0)),
        pl.BlockSpec((NC, _R, 16), lambda i: (0, i, 0)),
        pl.BlockSpec((_R, D), lambda i: (i, 0)),
        pl.BlockSpec((D, D), lambda i: (0, 0)),
        pl.BlockSpec((D, D), lambda i: (0, 0)),
        pl.BlockSpec((1, D), lambda i: (0, 0)),
    ]
    in_specs += [pl.BlockSpec((1, D), lambda i: (0, 0))] * n_extra
    return pl.pallas_call(
        body,
        grid=(N // _R,),
        in_specs=in_specs,
        out_specs=pl.BlockSpec((_R, D), lambda i: (i, 0)),
        out_shape=jax.ShapeDtypeStruct((N, D), jnp.float32),
    )


_dense_ln = _dense(_dense_ln_body, 2)
_dense_plain = _dense(_dense_plain_body, 0)


def kernel(x, edge_index, Wl0, bl0, Wr0, Wl1, bl1, Wr1, Wl2, bl2, Wr2,
           g0, b0, g1, b1):
    src = edge_index[0]
    dst = edge_index[1]
    pad = EPAD - E
    pad_dst = N + (jnp.arange(pad, dtype=jnp.int32) % (NP - N))
    srcp = jnp.concatenate(
        [src, jnp.zeros((pad,), jnp.int32)]).reshape(NW, CPW, CH)
    dstp = jnp.concatenate([dst, pad_dst]).reshape(NW, CPW, CH)

    def r(a):
        return a.reshape(1, D)

    p, cnt = _sc_segsum_cnt(x, srcp, dstp)
    x1 = _dense_ln(p, cnt, x, Wl0.T, Wr0.T, r(bl0), r(g0), r(b0))
    (p,) = _sc_segsum(x1, srcp, dstp)
    x2 = _dense_ln(p, cnt, x1, Wl1.T, Wr1.T, r(bl1), r(g1), r(b1))
    (p,) = _sc_segsum(x2, srcp, dstp)
    return _dense_plain(p, cnt, x2, Wl2.T, Wr2.T, r(bl2))


# speedup vs baseline: 8.5325x; 1.7058x over previous
import functools

import jax
import jax.numpy as jnp
from jax import lax
from jax.experimental import pallas as pl
from jax.experimental.pallas import tpu as pltpu
from jax.experimental.pallas import tpu_sc as plsc

N = 10000
D = 128
E = 320000

NC = 2
NS = 16
NW = NC * NS
CH = 64
CPW = 158
EPW = CH * CPW
EPAD = NW * EPW
NP = 10112
RPT = NP // NS
ZR = 64

_mesh = plsc.VectorSubcoreMesh(
    core_axis_name="c", subcore_axis_name="s", num_cores=NC, num_subcores=NS)


def _zero_rows16(ref, nrows):
    zv = jnp.zeros((16,), jnp.float32)

    def body(r, carry):
        ref[r] = zv
        return carry

    lax.fori_loop(0, nrows, body, 0)


def _make_sc_segsum(with_cnt):
    out_type = [jax.ShapeDtypeStruct((NC, NP, D), jnp.float32)]
    scratch = [
        pltpu.VMEM((CPW, CH), jnp.int32),
        pltpu.VMEM((CPW, CH), jnp.int32),
        pltpu.VMEM((2, CH, D), jnp.float32),
        pltpu.VMEM_SHARED((NP, D), jnp.float32),
        pltpu.SemaphoreType.DMA,
        pltpu.SemaphoreType.DMA,
    ]
    if with_cnt:
        out_type.append(jax.ShapeDtypeStruct((NC, NP, 16), jnp.float32))
        scratch += [
            pltpu.VMEM((ZR, 16), jnp.float32),
            pltpu.VMEM((CH, 16), jnp.float32),
            pltpu.VMEM_SHARED((NP, 16), jnp.float32),
        ]

    @functools.partial(
        pl.kernel, out_type=tuple(out_type), mesh=_mesh,
        scratch_types=tuple(scratch),
        compiler_params=pltpu.CompilerParams(use_tc_tiling_on_sc=False))
    def sc_segsum(x_hbm, srcs_hbm, dsts_hbm, *rest):
        if with_cnt:
            (out_hbm, cnt_hbm, src_v, dst_v, rows_v, acc_sh,
             sem0, sem1, zcnt, ones_v, cnt_sh) = rest
        else:
            (out_hbm, src_v, dst_v, rows_v, acc_sh, sem0, sem1) = rest

        core = lax.axis_index("c")
        sub = lax.axis_index("s")
        wid = sub * NC + core

        zv = jnp.zeros((16,), jnp.float32)

        def zrow(r, carry):
            def zcol(cc, carry2):
                rows_v[0, r, pl.ds(cc * 16, 16)] = zv
                return carry2
            return lax.fori_loop(0, D // 16, zcol, carry)

        lax.fori_loop(0, ZR, zrow, 0)
        if with_cnt:
            _zero_rows16(zcnt, ZR)
            ov = jnp.ones((16,), jnp.float32)

            def orow(r, carry):
                ones_v[r] = ov
                return carry

            lax.fori_loop(0, CH, orow, 0)

        pltpu.sync_copy(srcs_hbm.at[wid], src_v)
        pltpu.sync_copy(dsts_hbm.at[wid], dst_v)

        base = sub * RPT
        offs = [k * ZR for k in range(RPT // ZR)]
        if RPT % ZR:
            offs.append(RPT - ZR)
        for off in offs:
            pltpu.sync_copy(rows_v.at[0], acc_sh.at[pl.ds(base + off, ZR)])
            if with_cnt:
                pltpu.sync_copy(zcnt, cnt_sh.at[pl.ds(base + off, ZR)])
        plsc.subcore_barrier()

        sems = (sem0, sem1)
        pltpu.async_copy(x_hbm.at[src_v.at[0]], rows_v.at[0], sem0)

        def chunk(ci, buf):
            pltpu.make_async_copy(
                x_hbm.at[src_v.at[ci]], rows_v.at[buf], sems[buf]).wait()

            @pl.when(ci + 1 < CPW)
            def _():
                pltpu.async_copy(
                    x_hbm.at[src_v.at[ci + 1]], rows_v.at[1 - buf],
                    sems[1 - buf])

            pltpu.sync_copy(rows_v.at[buf], acc_sh.at[dst_v.at[ci]], add=True)
            if with_cnt:
                pltpu.sync_copy(ones_v, cnt_sh.at[dst_v.at[ci]], add=True)

        def body(g, carry):
            chunk(2 * g, 0)
            chunk(2 * g + 1, 1)
            return carry

        lax.fori_loop(0, CPW // 2, body, 0)
        plsc.subcore_barrier()

        pltpu.sync_copy(acc_sh.at[pl.ds(base, RPT)],
                        out_hbm.at[core, pl.ds(base, RPT)])
        if with_cnt:
            pltpu.sync_copy(cnt_sh.at[pl.ds(base, RPT)],
                            cnt_hbm.at[core, pl.ds(base, RPT)])

    return sc_segsum


_sc_segsum_cnt = _make_sc_segsum(True)
_sc_segsum = _make_sc_segsum(False)

_R = 400


def _dense_ln_body(p_ref, cnt_ref, x_ref, wl_ref, wr_ref, bl_ref, g_ref,
                   b_ref, o_ref):
    s = p_ref[0] + p_ref[1]
    c = cnt_ref[0, :, 0:1] + cnt_ref[1, :, 0:1]
    mean = s / jnp.maximum(c, 1.0)
    x = x_ref[...]
    h = jnp.dot(mean, wl_ref[...], preferred_element_type=jnp.float32)
    h = h + jnp.dot(x, wr_ref[...], preferred_element_type=jnp.float32)
    h = h + bl_ref[...]
    m = jnp.mean(h, axis=-1, keepdims=True)
    v = jnp.mean((h - m) * (h - m), axis=-1, keepdims=True)
    hn = (h - m) / jnp.sqrt(v + 1e-5) * g_ref[...] + b_ref[...]
    o_ref[...] = jnp.maximum(hn, 0.0) + x


def _dense_plain_body(p_ref, cnt_ref, x_ref, wl_ref, wr_ref, bl_ref, o_ref):
    s = p_ref[0] + p_ref[1]
    c = cnt_ref[0, :, 0:1] + cnt_ref[1, :, 0:1]
    mean = s / jnp.maximum(c, 1.0)
    h = jnp.dot(mean, wl_ref[...], preferred_element_type=jnp.float32)
    h = h + jnp.dot(x_ref[...], wr_ref[...], preferred_element_type=jnp.float32)
    o_ref[...] = h + bl_ref[...]


def _dense(body, n_extra):
    in_specs = [
        pl.BlockSpec((NC, _R, D), lambda i: (0, i, 0)),
        pl.BlockSpec((NC, _R, 16), lambda i: (0, i, 0)),
        pl.BlockSpec((_R, D), lambda i: (i, 0)),
        pl.BlockSpec((D, D), lambda i: (0, 0)),
        pl.BlockSpec((D, D), lambda i: (0, 0)),
        pl.BlockSpec((1, D), lambda i: (0, 0)),
    ]
    in_specs += [pl.BlockSpec((1, D), lambda i: (0, 0))] * n_extra
    return pl.pallas_call(
        body,
        grid=(N // _R,),
        in_specs=in_specs,
        out_specs=pl.BlockSpec((_R, D), lambda i: (i, 0)),
        out_shape=jax.ShapeDtypeStruct((N, D), jnp.float32),
    )


_dense_ln = _dense(_dense_ln_body, 2)
_dense_plain = _dense(_dense_plain_body, 0)


def kernel(x, edge_index, Wl0, bl0, Wr0, Wl1, bl1, Wr1, Wl2, bl2, Wr2,
           g0, b0, g1, b1):
    src = edge_index[0]
    dst = edge_index[1]
    pad = EPAD - E
    pad_src = jnp.arange(pad, dtype=jnp.int32) % N
    pad_dst = N + (jnp.arange(pad, dtype=jnp.int32) % (NP - N))
    srcp = jnp.concatenate([src, pad_src]).reshape(NW, CPW, CH)
    dstp = jnp.concatenate([dst, pad_dst]).reshape(NW, CPW, CH)

    def r(a):
        return a.reshape(1, D)

    p, cnt = _sc_segsum_cnt(x, srcp, dstp)
    x1 = _dense_ln(p, cnt, x, Wl0.T, Wr0.T, r(bl0), r(g0), r(b0))
    (p,) = _sc_segsum(x1, srcp, dstp)
    x2 = _dense_ln(p, cnt, x1, Wl1.T, Wr1.T, r(bl1), r(g1), r(b1))
    (p,) = _sc_segsum(x2, srcp, dstp)
    return _dense_plain(p, cnt, x2, Wl2.T, Wr2.T, r(bl2))


# speedup vs baseline: 9.9326x; 1.1641x over previous
import functools

import jax
import jax.numpy as jnp
from jax import lax
from jax.experimental import pallas as pl
from jax.experimental.pallas import tpu as pltpu
from jax.experimental.pallas import tpu_sc as plsc

N = 10000
D = 128
E = 320000

NC = 2
NS = 16
NW = NC * NS
NP = 10112
RPT = NP // NS
ZR = 64

_mesh = plsc.VectorSubcoreMesh(
    core_axis_name="c", subcore_axis_name="s", num_cores=NC, num_subcores=NS)


def _zero_rows16(ref, nrows):
    zv = jnp.zeros((16,), jnp.float32)

    def body(r, carry):
        ref[r] = zv
        return carry

    lax.fori_loop(0, nrows, body, 0)


def _make_sc_segsum(with_cnt, ch, cpw, nstages):
    spc = cpw // nstages
    out_type = [jax.ShapeDtypeStruct((NC, NP, D), jnp.float32)]
    scratch = [
        pltpu.VMEM((spc, ch), jnp.int32),
        pltpu.VMEM((spc, ch), jnp.int32),
        pltpu.VMEM((2, ch, D), jnp.float32),
        pltpu.VMEM_SHARED((NP, D), jnp.float32),
        pltpu.SemaphoreType.DMA,
        pltpu.SemaphoreType.DMA,
    ]
    if with_cnt:
        out_type.append(jax.ShapeDtypeStruct((NC, NP, 16), jnp.float32))
        scratch += [
            pltpu.VMEM((ZR, 16), jnp.float32),
            pltpu.VMEM((ch, 16), jnp.float32),
            pltpu.VMEM_SHARED((NP, 16), jnp.float32),
        ]

    @functools.partial(
        pl.kernel, out_type=tuple(out_type), mesh=_mesh,
        scratch_types=tuple(scratch),
        compiler_params=pltpu.CompilerParams(use_tc_tiling_on_sc=False))
    def sc_segsum(x_hbm, srcs_hbm, dsts_hbm, *rest):
        if with_cnt:
            (out_hbm, cnt_hbm, src_v, dst_v, rows_v, acc_sh,
             sem0, sem1, zcnt, ones_v, cnt_sh) = rest
        else:
            (out_hbm, src_v, dst_v, rows_v, acc_sh, sem0, sem1) = rest

        core = lax.axis_index("c")
        sub = lax.axis_index("s")
        wid = sub * NC + core

        zv = jnp.zeros((16,), jnp.float32)

        def zrow(r, carry):
            def zcol(cc, carry2):
                rows_v[0, r, pl.ds(cc * 16, 16)] = zv
                return carry2
            return lax.fori_loop(0, D // 16, zcol, carry)

        lax.fori_loop(0, ZR, zrow, 0)
        if with_cnt:
            _zero_rows16(zcnt, ZR)
            ov = jnp.ones((16,), jnp.float32)

            def orow(r, carry):
                ones_v[r] = ov
                return carry

            lax.fori_loop(0, ch, orow, 0)

        base = sub * RPT
        offs = [k * ZR for k in range(RPT // ZR)]
        if RPT % ZR:
            offs.append(RPT - ZR)
        zrows = rows_v.at[0, pl.ds(0, ZR)]
        for off in offs:
            pltpu.sync_copy(zrows, acc_sh.at[pl.ds(base + off, ZR)])
            if with_cnt:
                pltpu.sync_copy(zcnt, cnt_sh.at[pl.ds(base + off, ZR)])
        plsc.subcore_barrier()

        sems = (sem0, sem1)

        def chunk(ci, buf):
            pltpu.make_async_copy(
                x_hbm.at[src_v.at[ci]], rows_v.at[buf], sems[buf]).wait()

            @pl.when(ci + 1 < spc)
            def _():
                pltpu.async_copy(
                    x_hbm.at[src_v.at[ci + 1]], rows_v.at[1 - buf],
                    sems[1 - buf])

            pltpu.sync_copy(rows_v.at[buf], acc_sh.at[dst_v.at[ci]], add=True)
            if with_cnt:
                pltpu.sync_copy(ones_v, cnt_sh.at[dst_v.at[ci]], add=True)

        def group(g, carry):
            chunk(2 * g, 0)
            chunk(2 * g + 1, 1)
            return carry

        for s in range(nstages):
            pltpu.sync_copy(srcs_hbm.at[wid, pl.ds(s * spc, spc)], src_v)
            pltpu.sync_copy(dsts_hbm.at[wid, pl.ds(s * spc, spc)], dst_v)
            pltpu.async_copy(x_hbm.at[src_v.at[0]], rows_v.at[0], sem0)
            lax.fori_loop(0, spc // 2, group, 0)
        plsc.subcore_barrier()

        pltpu.sync_copy(acc_sh.at[pl.ds(base, RPT)],
                        out_hbm.at[core, pl.ds(base, RPT)])
        if with_cnt:
            pltpu.sync_copy(cnt_sh.at[pl.ds(base, RPT)],
                            cnt_hbm.at[core, pl.ds(base, RPT)])

    return sc_segsum


CH0, CPW0, NST0 = 64, 158, 1
CH1, CPW1, NST1 = 128, 80, 2
_sc_segsum_cnt = _make_sc_segsum(True, CH0, CPW0, NST0)
_sc_segsum = _make_sc_segsum(False, CH1, CPW1, NST1)


def _pad_edges(src, dst, ch, cpw):
    pad = NW * cpw * ch - E
    pad_src = jnp.arange(pad, dtype=jnp.int32) % N
    pad_dst = N + (jnp.arange(pad, dtype=jnp.int32) % (NP - N))
    srcp = jnp.concatenate([src, pad_src]).reshape(NW, cpw, ch)
    dstp = jnp.concatenate([dst, pad_dst]).reshape(NW, cpw, ch)
    return srcp, dstp

_R = 400


def _dense_ln_body(p_ref, cnt_ref, x_ref, wl_ref, wr_ref, bl_ref, g_ref,
                   b_ref, o_ref):
    s = p_ref[0] + p_ref[1]
    c = cnt_ref[0, :, 0:1] + cnt_ref[1, :, 0:1]
    mean = s / jnp.maximum(c, 1.0)
    x = x_ref[...]
    h = jnp.dot(mean, wl_ref[...], preferred_element_type=jnp.float32)
    h = h + jnp.dot(x, wr_ref[...], preferred_element_type=jnp.float32)
    h = h + bl_ref[...]
    m = jnp.mean(h, axis=-1, keepdims=True)
    v = jnp.mean((h - m) * (h - m), axis=-1, keepdims=True)
    hn = (h - m) / jnp.sqrt(v + 1e-5) * g_ref[...] + b_ref[...]
    o_ref[...] = jnp.maximum(hn, 0.0) + x


def _dense_plain_body(p_ref, cnt_ref, x_ref, wl_ref, wr_ref, bl_ref, o_ref):
    s = p_ref[0] + p_ref[1]
    c = cnt_ref[0, :, 0:1] + cnt_ref[1, :, 0:1]
    mean = s / jnp.maximum(c, 1.0)
    h = jnp.dot(mean, wl_ref[...], preferred_element_type=jnp.float32)
    h = h + jnp.dot(x_ref[...], wr_ref[...], preferred_element_type=jnp.float32)
    o_ref[...] = h + bl_ref[...]


def _dense(body, n_extra):
    in_specs = [
        pl.BlockSpec((NC, _R, D), lambda i: (0, i, 0)),
        pl.BlockSpec((NC, _R, 16), lambda i: (0, i, 0)),
        pl.BlockSpec((_R, D), lambda i: (i, 0)),
        pl.BlockSpec((D, D), lambda i: (0, 0)),
        pl.BlockSpec((D, D), lambda i: (0, 0)),
        pl.BlockSpec((1, D), lambda i: (0, 0)),
    ]
    in_specs += [pl.BlockSpec((1, D), lambda i: (0, 0))] * n_extra
    return pl.pallas_call(
        body,
        grid=(N // _R,),
        in_specs=in_specs,
        out_specs=pl.BlockSpec((_R, D), lambda i: (i, 0)),
        out_shape=jax.ShapeDtypeStruct((N, D), jnp.float32),
    )


_dense_ln = _dense(_dense_ln_body, 2)
_dense_plain = _dense(_dense_plain_body, 0)


def kernel(x, edge_index, Wl0, bl0, Wr0, Wl1, bl1, Wr1, Wl2, bl2, Wr2,
           g0, b0, g1, b1):
    src = edge_index[0]
    dst = edge_index[1]
    srcp0, dstp0 = _pad_edges(src, dst, CH0, CPW0)
    srcp1, dstp1 = _pad_edges(src, dst, CH1, CPW1)

    def r(a):
        return a.reshape(1, D)

    p, cnt = _sc_segsum_cnt(x, srcp0, dstp0)
    x1 = _dense_ln(p, cnt, x, Wl0.T, Wr0.T, r(bl0), r(g0), r(b0))
    (p,) = _sc_segsum(x1, srcp1, dstp1)
    x2 = _dense_ln(p, cnt, x1, Wl1.T, Wr1.T, r(bl1), r(g1), r(b1))
    (p,) = _sc_segsum(x2, srcp1, dstp1)
    return _dense_plain(p, cnt, x2, Wl2.T, Wr2.T, r(bl2))


# speedup vs baseline: 10.6252x; 1.0697x over previous
import functools

import jax
import jax.numpy as jnp
from jax import lax
from jax.experimental import pallas as pl
from jax.experimental.pallas import tpu as pltpu
from jax.experimental.pallas import tpu_sc as plsc

N = 10000
D = 128
E = 320000

NC = 2
NS = 16
NW = NC * NS
NP = 10112
RPT = NP // NS
ZR = 64

_mesh = plsc.VectorSubcoreMesh(
    core_axis_name="c", subcore_axis_name="s", num_cores=NC, num_subcores=NS)


def _zero_rows16(ref, nrows):
    zv = jnp.zeros((16,), jnp.float32)

    def body(r, carry):
        ref[r] = zv
        return carry

    lax.fori_loop(0, nrows, body, 0)


def _make_sc_segsum(with_cnt, ch, cpw, nstages):
    spc = cpw // nstages
    out_type = [jax.ShapeDtypeStruct((NC, NP, D), jnp.float32)]
    scratch = [
        pltpu.VMEM((spc, ch), jnp.int32),
        pltpu.VMEM((spc, ch), jnp.int32),
        pltpu.VMEM((2, ch, D), jnp.float32),
        pltpu.VMEM_SHARED((NP, D), jnp.float32),
        pltpu.SemaphoreType.DMA,
        pltpu.SemaphoreType.DMA,
    ]
    if with_cnt:
        out_type.append(jax.ShapeDtypeStruct((NC, NP, 16), jnp.float32))
        scratch += [
            pltpu.VMEM((ZR, 16), jnp.float32),
            pltpu.VMEM((ch, 16), jnp.float32),
            pltpu.VMEM_SHARED((NP, 16), jnp.float32),
        ]

    @functools.partial(
        pl.kernel, out_type=tuple(out_type), mesh=_mesh,
        scratch_types=tuple(scratch),
        compiler_params=pltpu.CompilerParams(use_tc_tiling_on_sc=False))
    def sc_segsum(x_hbm, srcs_hbm, dsts_hbm, *rest):
        if with_cnt:
            (out_hbm, cnt_hbm, src_v, dst_v, rows_v, acc_sh,
             sem0, sem1, zcnt, ones_v, cnt_sh) = rest
        else:
            (out_hbm, src_v, dst_v, rows_v, acc_sh, sem0, sem1) = rest

        core = lax.axis_index("c")
        sub = lax.axis_index("s")
        wid = sub * NC + core

        zv = jnp.zeros((16,), jnp.float32)

        def zrow(r, carry):
            def zcol(cc, carry2):
                rows_v[0, r, pl.ds(cc * 16, 16)] = zv
                return carry2
            return lax.fori_loop(0, D // 16, zcol, carry)

        lax.fori_loop(0, ZR, zrow, 0)
        if with_cnt:
            _zero_rows16(zcnt, ZR)
            ov = jnp.ones((16,), jnp.float32)

            def orow(r, carry):
                ones_v[r] = ov
                return carry

            lax.fori_loop(0, ch, orow, 0)

        base = sub * RPT
        offs = [k * ZR for k in range(RPT // ZR)]
        if RPT % ZR:
            offs.append(RPT - ZR)
        zrows = rows_v.at[0, pl.ds(0, ZR)]
        for off in offs:
            pltpu.sync_copy(zrows, acc_sh.at[pl.ds(base + off, ZR)])
            if with_cnt:
                pltpu.sync_copy(zcnt, cnt_sh.at[pl.ds(base + off, ZR)])
        plsc.subcore_barrier()

        sems = (sem0, sem1)

        def chunk(ci, buf):
            pltpu.make_async_copy(
                x_hbm.at[src_v.at[ci]], rows_v.at[buf], sems[buf]).wait()

            @pl.when(ci + 1 < spc)
            def _():
                pltpu.async_copy(
                    x_hbm.at[src_v.at[ci + 1]], rows_v.at[1 - buf],
                    sems[1 - buf])

            pltpu.sync_copy(rows_v.at[buf], acc_sh.at[dst_v.at[ci]], add=True)
            if with_cnt:
                pltpu.sync_copy(ones_v, cnt_sh.at[dst_v.at[ci]], add=True)

        def group(g, carry):
            chunk(2 * g, 0)
            chunk(2 * g + 1, 1)
            return carry

        for s in range(nstages):
            pltpu.sync_copy(srcs_hbm.at[wid, pl.ds(s * spc, spc)], src_v)
            pltpu.sync_copy(dsts_hbm.at[wid, pl.ds(s * spc, spc)], dst_v)
            pltpu.async_copy(x_hbm.at[src_v.at[0]], rows_v.at[0], sem0)
            lax.fori_loop(0, spc // 2, group, 0)
        plsc.subcore_barrier()

        pltpu.sync_copy(acc_sh.at[pl.ds(base, RPT)],
                        out_hbm.at[core, pl.ds(base, RPT)])
        if with_cnt:
            pltpu.sync_copy(cnt_sh.at[pl.ds(base, RPT)],
                            cnt_hbm.at[core, pl.ds(base, RPT)])

    return sc_segsum


CH0, CPW0, NST0 = 128, 80, 5
CH1, CPW1, NST1 = 128, 80, 2
_sc_segsum_cnt = _make_sc_segsum(True, CH0, CPW0, NST0)
_sc_segsum = _make_sc_segsum(False, CH1, CPW1, NST1)


def _pad_edges(src, dst, ch, cpw):
    pad = NW * cpw * ch - E
    pad_src = jnp.arange(pad, dtype=jnp.int32) % N
    pad_dst = N + (jnp.arange(pad, dtype=jnp.int32) % (NP - N))
    srcp = jnp.concatenate([src, pad_src]).reshape(NW, cpw, ch)
    dstp = jnp.concatenate([dst, pad_dst]).reshape(NW, cpw, ch)
    return srcp, dstp

_R = 400


def _dot_t(a, w):
    return jax.lax.dot_general(
        a, w, (((1,), (1,)), ((), ())),
        preferred_element_type=jnp.float32)


def _dense_ln_body(p_ref, cnt_ref, x_ref, wl_ref, wr_ref, bl_ref, g_ref,
                   b_ref, o_ref):
    s = p_ref[0] + p_ref[1]
    c = cnt_ref[0, :, 0:1] + cnt_ref[1, :, 0:1]
    mean = s / jnp.maximum(c, 1.0)
    x = x_ref[...]
    h = _dot_t(mean, wl_ref[...]) + _dot_t(x, wr_ref[...]) + bl_ref[...]
    m = jnp.mean(h, axis=-1, keepdims=True)
    v = jnp.mean((h - m) * (h - m), axis=-1, keepdims=True)
    hn = (h - m) / jnp.sqrt(v + 1e-5) * g_ref[...] + b_ref[...]
    o_ref[...] = jnp.maximum(hn, 0.0) + x


def _dense_plain_body(p_ref, cnt_ref, x_ref, wl_ref, wr_ref, bl_ref, o_ref):
    s = p_ref[0] + p_ref[1]
    c = cnt_ref[0, :, 0:1] + cnt_ref[1, :, 0:1]
    mean = s / jnp.maximum(c, 1.0)
    h = _dot_t(mean, wl_ref[...]) + _dot_t(x_ref[...], wr_ref[...])
    o_ref[...] = h + bl_ref[...]


def _dense(body, n_extra):
    in_specs = [
        pl.BlockSpec((NC, _R, D), lambda i: (0, i, 0)),
        pl.BlockSpec((NC, _R, 16), lambda i: (0, i, 0)),
        pl.BlockSpec((_R, D), lambda i: (i, 0)),
        pl.BlockSpec((D, D), lambda i: (0, 0)),
        pl.BlockSpec((D, D), lambda i: (0, 0)),
        pl.BlockSpec((1, D), lambda i: (0, 0)),
    ]
    in_specs += [pl.BlockSpec((1, D), lambda i: (0, 0))] * n_extra
    return pl.pallas_call(
        body,
        grid=(N // _R,),
        in_specs=in_specs,
        out_specs=pl.BlockSpec((_R, D), lambda i: (i, 0)),
        out_shape=jax.ShapeDtypeStruct((N, D), jnp.float32),
    )


_dense_ln = _dense(_dense_ln_body, 2)
_dense_plain = _dense(_dense_plain_body, 0)


def kernel(x, edge_index, Wl0, bl0, Wr0, Wl1, bl1, Wr1, Wl2, bl2, Wr2,
           g0, b0, g1, b1):
    src = edge_index[0]
    dst = edge_index[1]
    srcp0, dstp0 = _pad_edges(src, dst, CH0, CPW0)
    srcp1, dstp1 = srcp0, dstp0

    def r(a):
        return a.reshape(1, D)

    p, cnt = _sc_segsum_cnt(x, srcp0, dstp0)
    x1 = _dense_ln(p, cnt, x, Wl0, Wr0, r(bl0), r(g0), r(b0))
    (p,) = _sc_segsum(x1, srcp1, dstp1)
    x2 = _dense_ln(p, cnt, x1, Wl1, Wr1, r(bl1), r(g1), r(b1))
    (p,) = _sc_segsum(x2, srcp1, dstp1)
    return _dense_plain(p, cnt, x2, Wl2, Wr2, r(bl2))


# speedup vs baseline: 11.2351x; 1.0574x over previous
import functools

import jax
import jax.numpy as jnp
from jax import lax
from jax.experimental import pallas as pl
from jax.experimental.pallas import tpu as pltpu
from jax.experimental.pallas import tpu_sc as plsc

N = 10000
D = 128
E = 320000

NC = 2
NS = 16
NW = NC * NS
NP = 10112
RPT = NP // NS
ZR = 64

_mesh = plsc.VectorSubcoreMesh(
    core_axis_name="c", subcore_axis_name="s", num_cores=NC, num_subcores=NS)


def _zero_rows16(ref, nrows):
    zv = jnp.zeros((16,), jnp.float32)

    def body(r, carry):
        ref[r] = zv
        return carry

    lax.fori_loop(0, nrows, body, 0)


def _make_sc_segsum(with_cnt, ch, cpw, nstages):
    spc = cpw // nstages
    out_type = [jax.ShapeDtypeStruct((NC, NP, D), jnp.float32)]
    scratch = [
        pltpu.VMEM((spc, ch), jnp.int32),
        pltpu.VMEM((spc, ch), jnp.int32),
        pltpu.VMEM((2, ch, D), jnp.float32),
        pltpu.VMEM_SHARED((NP, D), jnp.float32),
        pltpu.SemaphoreType.DMA,
        pltpu.SemaphoreType.DMA,
    ]
    if with_cnt:
        out_type.append(jax.ShapeDtypeStruct((NC, NP, 16), jnp.float32))
        scratch += [
            pltpu.VMEM((ZR, 16), jnp.float32),
            pltpu.VMEM((ch, 16), jnp.float32),
            pltpu.VMEM_SHARED((NP, 16), jnp.float32),
        ]

    @functools.partial(
        pl.kernel, out_type=tuple(out_type), mesh=_mesh,
        scratch_types=tuple(scratch),
        compiler_params=pltpu.CompilerParams(use_tc_tiling_on_sc=False))
    def sc_segsum(x_hbm, edges_hbm, *rest):
        if with_cnt:
            (out_hbm, cnt_hbm, src_v, dst_v, rows_v, acc_sh,
             sem0, sem1, zcnt, ones_v, cnt_sh) = rest
        else:
            (out_hbm, src_v, dst_v, rows_v, acc_sh, sem0, sem1) = rest

        core = lax.axis_index("c")
        sub = lax.axis_index("s")
        wid = sub * NC + core

        zv = jnp.zeros((16,), jnp.float32)

        def zrow(r, carry):
            def zcol(cc, carry2):
                rows_v[0, r, pl.ds(cc * 16, 16)] = zv
                return carry2
            return lax.fori_loop(0, D // 16, zcol, carry)

        lax.fori_loop(0, ZR, zrow, 0)
        if with_cnt:
            _zero_rows16(zcnt, ZR)
            ov = jnp.ones((16,), jnp.float32)

            def orow(r, carry):
                ones_v[r] = ov
                return carry

            lax.fori_loop(0, ch, orow, 0)

        base = sub * RPT
        offs = [k * ZR for k in range(RPT // ZR)]
        if RPT % ZR:
            offs.append(RPT - ZR)
        zrows = rows_v.at[0, pl.ds(0, ZR)]
        for off in offs:
            pltpu.sync_copy(zrows, acc_sh.at[pl.ds(base + off, ZR)])
            if with_cnt:
                pltpu.sync_copy(zcnt, cnt_sh.at[pl.ds(base + off, ZR)])
        plsc.subcore_barrier()

        sems = (sem0, sem1)

        def chunk(ci, buf):
            pltpu.make_async_copy(
                x_hbm.at[src_v.at[ci]], rows_v.at[buf], sems[buf]).wait()

            @pl.when(ci + 1 < spc)
            def _():
                pltpu.async_copy(
                    x_hbm.at[src_v.at[ci + 1]], rows_v.at[1 - buf],
                    sems[1 - buf])

            pltpu.sync_copy(rows_v.at[buf], acc_sh.at[dst_v.at[ci]], add=True)
            if with_cnt:
                pltpu.sync_copy(ones_v, cnt_sh.at[dst_v.at[ci]], add=True)

        def group(g, carry):
            chunk(2 * g, 0)
            chunk(2 * g + 1, 1)
            return carry

        for s in range(nstages):
            pltpu.sync_copy(edges_hbm.at[0, wid, pl.ds(s * spc, spc)], src_v)
            pltpu.sync_copy(edges_hbm.at[1, wid, pl.ds(s * spc, spc)], dst_v)
            pltpu.async_copy(x_hbm.at[src_v.at[0]], rows_v.at[0], sem0)
            lax.fori_loop(0, spc // 2, group, 0)
        plsc.subcore_barrier()

        pltpu.sync_copy(acc_sh.at[pl.ds(base, RPT)],
                        out_hbm.at[core, pl.ds(base, RPT)])
        if with_cnt:
            pltpu.sync_copy(cnt_sh.at[pl.ds(base, RPT)],
                            cnt_hbm.at[core, pl.ds(base, RPT)])

    return sc_segsum


CH0, CPW0, NST0 = 128, 80, 5
CH1, CPW1, NST1 = 128, 80, 2
_sc_segsum_cnt = _make_sc_segsum(True, CH0, CPW0, NST0)
_sc_segsum = _make_sc_segsum(False, CH1, CPW1, NST1)


def _pad_edges(edge_index, ch, cpw):
    pad = NW * cpw * ch - E
    ar = jnp.arange(pad, dtype=jnp.int32)
    pad_blk = jnp.stack([ar % N, N + ar % (NP - N)])
    return jnp.concatenate([edge_index, pad_blk], axis=1).reshape(
        2, NW, cpw, ch)

_R = 1000


def _dot_t(a, w):
    return jax.lax.dot_general(
        a, w, (((1,), (1,)), ((), ())),
        preferred_element_type=jnp.float32)


def _mean_rows(p_ref, cnt_ref):
    i = pl.program_id(0)
    s = p_ref[0] + p_ref[1]
    c = (cnt_ref[0, pl.ds(i * _R, _R), 0:1]
         + cnt_ref[1, pl.ds(i * _R, _R), 0:1])
    return s / jnp.maximum(c, 1.0)


def _dense_ln_body(p_ref, cnt_ref, x_ref, wl_ref, wr_ref, bl_ref, g_ref,
                   b_ref, o_ref):
    mean = _mean_rows(p_ref, cnt_ref)
    x = x_ref[...]
    h = _dot_t(mean, wl_ref[...]) + _dot_t(x, wr_ref[...]) + bl_ref[...]
    m = jnp.mean(h, axis=-1, keepdims=True)
    v = jnp.mean((h - m) * (h - m), axis=-1, keepdims=True)
    hn = (h - m) / jnp.sqrt(v + 1e-5) * g_ref[...] + b_ref[...]
    o_ref[...] = jnp.maximum(hn, 0.0) + x


def _dense_plain_body(p_ref, cnt_ref, x_ref, wl_ref, wr_ref, bl_ref, o_ref):
    mean = _mean_rows(p_ref, cnt_ref)
    h = _dot_t(mean, wl_ref[...]) + _dot_t(x_ref[...], wr_ref[...])
    o_ref[...] = h + bl_ref[...]


def _dense(body, n_extra):
    in_specs = [
        pl.BlockSpec((NC, _R, D), lambda i: (0, i, 0)),
        pl.BlockSpec((NC, NP, 16), lambda i: (0, 0, 0)),
        pl.BlockSpec((_R, D), lambda i: (i, 0)),
        pl.BlockSpec((D, D), lambda i: (0, 0)),
        pl.BlockSpec((D, D), lambda i: (0, 0)),
        pl.BlockSpec((1, D), lambda i: (0, 0)),
    ]
    in_specs += [pl.BlockSpec((1, D), lambda i: (0, 0))] * n_extra
    return pl.pallas_call(
        body,
        grid=(N // _R,),
        in_specs=in_specs,
        out_specs=pl.BlockSpec((_R, D), lambda i: (i, 0)),
        out_shape=jax.ShapeDtypeStruct((N, D), jnp.float32),
    )


_dense_ln = _dense(_dense_ln_body, 2)
_dense_plain = _dense(_dense_plain_body, 0)


def kernel(x, edge_index, Wl0, bl0, Wr0, Wl1, bl1, Wr1, Wl2, bl2, Wr2,
           g0, b0, g1, b1):
    edges = _pad_edges(edge_index, CH0, CPW0)

    def r(a):
        return a.reshape(1, D)

    p, cnt = _sc_segsum_cnt(x, edges)
    x1 = _dense_ln(p, cnt, x, Wl0, Wr0, r(bl0), r(g0), r(b0))
    (p,) = _sc_segsum(x1, edges)
    x2 = _dense_ln(p, cnt, x1, Wl1, Wr1, r(bl1), r(g1), r(b1))
    (p,) = _sc_segsum(x2, edges)
    return _dense_plain(p, cnt, x2, Wl2, Wr2, r(bl2))


# speedup vs baseline: 11.2822x; 1.0042x over previous
import functools

import jax
import jax.numpy as jnp
from jax import lax
from jax.experimental import pallas as pl
from jax.experimental.pallas import tpu as pltpu
from jax.experimental.pallas import tpu_sc as plsc

N = 10000
D = 128
E = 320000

NC = 2
NS = 16
NW = NC * NS
NP = 10112
RPT = NP // NS
ZR = 64

_mesh = plsc.VectorSubcoreMesh(
    core_axis_name="c", subcore_axis_name="s", num_cores=NC, num_subcores=NS)


def _zero_rows16(ref, nrows):
    zv = jnp.zeros((16,), jnp.float32)

    def body(r, carry):
        ref[r] = zv
        return carry

    lax.fori_loop(0, nrows, body, 0)


def _make_sc_segsum(with_cnt, ch, cpw, nstages):
    spc = cpw // nstages
    out_type = [jax.ShapeDtypeStruct((NC, NP, D), jnp.float32)]
    scratch = [
        pltpu.VMEM((spc, ch), jnp.int32),
        pltpu.VMEM((spc, ch), jnp.int32),
        pltpu.VMEM((2, ch, D), jnp.float32),
        pltpu.VMEM_SHARED((NP, D), jnp.float32),
        pltpu.SemaphoreType.DMA,
        pltpu.SemaphoreType.DMA,
    ]
    if with_cnt:
        out_type.append(jax.ShapeDtypeStruct((NC, NP, 16), jnp.float32))
        scratch += [
            pltpu.VMEM((ZR, 16), jnp.float32),
            pltpu.VMEM((ch, 16), jnp.float32),
            pltpu.VMEM_SHARED((NP, 16), jnp.float32),
        ]

    @functools.partial(
        pl.kernel, out_type=tuple(out_type), mesh=_mesh,
        scratch_types=tuple(scratch),
        compiler_params=pltpu.CompilerParams(use_tc_tiling_on_sc=False))
    def sc_segsum(x_hbm, edges_hbm, *rest):
        if with_cnt:
            (out_hbm, cnt_hbm, src_v, dst_v, rows_v, acc_sh,
             sem0, sem1, zcnt, ones_v, cnt_sh) = rest
        else:
            (out_hbm, src_v, dst_v, rows_v, acc_sh, sem0, sem1) = rest

        core = lax.axis_index("c")
        sub = lax.axis_index("s")
        wid = sub * NC + core

        zv = jnp.zeros((16,), jnp.float32)

        def zrow(r, carry):
            def zcol(cc, carry2):
                rows_v[0, r, pl.ds(cc * 16, 16)] = zv
                return carry2
            return lax.fori_loop(0, D // 16, zcol, carry)

        lax.fori_loop(0, ZR, zrow, 0)
        if with_cnt:
            _zero_rows16(zcnt, ZR)
            ov = jnp.ones((16,), jnp.float32)

            def orow(r, carry):
                ones_v[r] = ov
                return carry

            lax.fori_loop(0, ch, orow, 0)

        base = sub * RPT
        offs = [k * ZR for k in range(RPT // ZR)]
        if RPT % ZR:
            offs.append(RPT - ZR)
        zrows = rows_v.at[0, pl.ds(0, ZR)]
        for off in offs:
            pltpu.sync_copy(zrows, acc_sh.at[pl.ds(base + off, ZR)])
            if with_cnt:
                pltpu.sync_copy(zcnt, cnt_sh.at[pl.ds(base + off, ZR)])
        plsc.subcore_barrier()

        sems = (sem0, sem1)

        def chunk(ci, buf):
            pltpu.make_async_copy(
                x_hbm.at[src_v.at[ci]], rows_v.at[buf], sems[buf]).wait()

            @pl.when(ci + 1 < spc)
            def _():
                pltpu.async_copy(
                    x_hbm.at[src_v.at[ci + 1]], rows_v.at[1 - buf],
                    sems[1 - buf])

            pltpu.sync_copy(rows_v.at[buf], acc_sh.at[dst_v.at[ci]], add=True)
            if with_cnt:
                pltpu.sync_copy(ones_v, cnt_sh.at[dst_v.at[ci]], add=True)

        def group(g, carry):
            chunk(2 * g, 0)
            chunk(2 * g + 1, 1)
            return carry

        for s in range(nstages):
            pltpu.sync_copy(edges_hbm.at[0, wid, pl.ds(s * spc, spc)], src_v)
            pltpu.sync_copy(edges_hbm.at[1, wid, pl.ds(s * spc, spc)], dst_v)
            pltpu.async_copy(x_hbm.at[src_v.at[0]], rows_v.at[0], sem0)
            lax.fori_loop(0, spc // 2, group, 0)
        plsc.subcore_barrier()

        pltpu.sync_copy(acc_sh.at[pl.ds(base, RPT)],
                        out_hbm.at[core, pl.ds(base, RPT)])
        if with_cnt:
            pltpu.sync_copy(cnt_sh.at[pl.ds(base, RPT)],
                            cnt_hbm.at[core, pl.ds(base, RPT)])

    return sc_segsum


CH0, CPW0, NST0 = 128, 80, 5
CH1, CPW1, NST1 = 128, 80, 2
_sc_segsum_cnt = _make_sc_segsum(True, CH0, CPW0, NST0)
_sc_segsum = _make_sc_segsum(False, CH1, CPW1, NST1)


def _pad_edges(edge_index, ch, cpw):
    pad = NW * cpw * ch - E
    ar = jnp.arange(pad, dtype=jnp.int32)
    pad_blk = jnp.stack([ar % N, N + ar % (NP - N)])
    return jnp.concatenate([edge_index, pad_blk], axis=1).reshape(
        2, NW, cpw, ch)

_R = 1000


def _dot_t(a, w):
    return jax.lax.dot_general(
        a.astype(jnp.bfloat16), w.astype(jnp.bfloat16),
        (((1,), (1,)), ((), ())),
        preferred_element_type=jnp.float32)


def _mean_rows(p_ref, cnt_ref):
    i = pl.program_id(0)
    s = p_ref[0] + p_ref[1]
    c = (cnt_ref[0, pl.ds(i * _R, _R), 0:1]
         + cnt_ref[1, pl.ds(i * _R, _R), 0:1])
    return s / jnp.maximum(c, 1.0)


def _dense_ln_body(p_ref, cnt_ref, x_ref, wl_ref, wr_ref, bl_ref, g_ref,
                   b_ref, o_ref):
    mean = _mean_rows(p_ref, cnt_ref)
    x = x_ref[...]
    h = _dot_t(mean, wl_ref[...]) + _dot_t(x, wr_ref[...]) + bl_ref[...]
    m = jnp.mean(h, axis=-1, keepdims=True)
    v = jnp.mean((h - m) * (h - m), axis=-1, keepdims=True)
    hn = (h - m) / jnp.sqrt(v + 1e-5) * g_ref[...] + b_ref[...]
    o_ref[...] = jnp.maximum(hn, 0.0) + x


def _dense_plain_body(p_ref, cnt_ref, x_ref, wl_ref, wr_ref, bl_ref, o_ref):
    mean = _mean_rows(p_ref, cnt_ref)
    h = _dot_t(mean, wl_ref[...]) + _dot_t(x_ref[...], wr_ref[...])
    o_ref[...] = h + bl_ref[...]


def _dense(body, n_extra):
    in_specs = [
        pl.BlockSpec((NC, _R, D), lambda i: (0, i, 0)),
        pl.BlockSpec((NC, NP, 16), lambda i: (0, 0, 0)),
        pl.BlockSpec((_R, D), lambda i: (i, 0)),
        pl.BlockSpec((D, D), lambda i: (0, 0)),
        pl.BlockSpec((D, D), lambda i: (0, 0)),
        pl.BlockSpec((1, D), lambda i: (0, 0)),
    ]
    in_specs += [pl.BlockSpec((1, D), lambda i: (0, 0))] * n_extra
    return pl.pallas_call(
        body,
        grid=(N // _R,),
        in_specs=in_specs,
        out_specs=pl.BlockSpec((_R, D), lambda i: (i, 0)),
        out_shape=jax.ShapeDtypeStruct((N, D), jnp.float32),
    )


_dense_ln = _dense(_dense_ln_body, 2)
_dense_plain = _dense(_dense_plain_body, 0)


def kernel(x, edge_index, Wl0, bl0, Wr0, Wl1, bl1, Wr1, Wl2, bl2, Wr2,
           g0, b0, g1, b1):
    edges = _pad_edges(edge_index, CH0, CPW0)

    def r(a):
        return a.reshape(1, D)

    p, cnt = _sc_segsum_cnt(x, edges)
    x1 = _dense_ln(p, cnt, x, Wl0, Wr0, r(bl0), r(g0), r(b0))
    (p,) = _sc_segsum(x1, edges)
    x2 = _dense_ln(p, cnt, x1, Wl1, Wr1, r(bl1), r(g1), r(b1))
    (p,) = _sc_segsum(x2, edges)
    return _dense_plain(p, cnt, x2, Wl2, Wr2, r(bl2))


# speedup vs baseline: 11.3915x; 1.0097x over previous
import functools

import jax
import jax.numpy as jnp
from jax import lax
from jax.experimental import pallas as pl
from jax.experimental.pallas import tpu as pltpu
from jax.experimental.pallas import tpu_sc as plsc

N = 10000
D = 128
E = 320000

NC = 2
NS = 16
NW = NC * NS
NP = 10016
RPT = NP // NS
ZR = 64

_mesh = plsc.VectorSubcoreMesh(
    core_axis_name="c", subcore_axis_name="s", num_cores=NC, num_subcores=NS)


def _zero_rows16(ref, nrows):
    zv = jnp.zeros((16,), jnp.float32)

    def body(r, carry):
        ref[r] = zv
        return carry

    lax.fori_loop(0, nrows, body, 0)


def _make_sc_segsum(with_cnt, ch, cpw, nstages):
    spc = cpw // nstages
    out_type = [jax.ShapeDtypeStruct((NC, NP, D), jnp.float32)]
    scratch = [
        pltpu.VMEM((spc, ch), jnp.int32),
        pltpu.VMEM((spc, ch), jnp.int32),
        pltpu.VMEM((2, ch, D), jnp.float32),
        pltpu.VMEM_SHARED((NP, D), jnp.float32),
        pltpu.SemaphoreType.DMA,
        pltpu.SemaphoreType.DMA,
    ]
    if with_cnt:
        out_type.append(jax.ShapeDtypeStruct((NC, NP, 16), jnp.float32))
        scratch += [
            pltpu.VMEM((ZR, 16), jnp.float32),
            pltpu.VMEM((ch, 16), jnp.float32),
            pltpu.VMEM_SHARED((NP, 16), jnp.float32),
        ]

    @functools.partial(
        pl.kernel, out_type=tuple(out_type), mesh=_mesh,
        scratch_types=tuple(scratch),
        compiler_params=pltpu.CompilerParams(use_tc_tiling_on_sc=False))
    def sc_segsum(x_hbm, edges_hbm, *rest):
        if with_cnt:
            (out_hbm, cnt_hbm, src_v, dst_v, rows_v, acc_sh,
             sem0, sem1, zcnt, ones_v, cnt_sh) = rest
        else:
            (out_hbm, src_v, dst_v, rows_v, acc_sh, sem0, sem1) = rest

        core = lax.axis_index("c")
        sub = lax.axis_index("s")
        wid = sub * NC + core

        zv = jnp.zeros((16,), jnp.float32)

        def zrow(r, carry):
            def zcol(cc, carry2):
                rows_v[0, r, pl.ds(cc * 16, 16)] = zv
                return carry2
            return lax.fori_loop(0, D // 16, zcol, carry)

        lax.fori_loop(0, ZR, zrow, 0)
        if with_cnt:
            _zero_rows16(zcnt, ZR)
            ov = jnp.ones((16,), jnp.float32)

            def orow(r, carry):
                ones_v[r] = ov
                return carry

            lax.fori_loop(0, ch, orow, 0)

        base = sub * RPT
        offs = [k * ZR for k in range(RPT // ZR)]
        if RPT % ZR:
            offs.append(RPT - ZR)
        zrows = rows_v.at[0, pl.ds(0, ZR)]
        for off in offs:
            pltpu.sync_copy(zrows, acc_sh.at[pl.ds(base + off, ZR)])
            if with_cnt:
                pltpu.sync_copy(zcnt, cnt_sh.at[pl.ds(base + off, ZR)])
        plsc.subcore_barrier()

        sems = (sem0, sem1)

        def chunk(ci, buf):
            pltpu.make_async_copy(
                x_hbm.at[src_v.at[ci]], rows_v.at[buf], sems[buf]).wait()

            @pl.when(ci + 1 < spc)
            def _():
                pltpu.async_copy(
                    x_hbm.at[src_v.at[ci + 1]], rows_v.at[1 - buf],
                    sems[1 - buf])

            pltpu.sync_copy(rows_v.at[buf], acc_sh.at[dst_v.at[ci]], add=True)
            if with_cnt:
                pltpu.sync_copy(ones_v, cnt_sh.at[dst_v.at[ci]], add=True)

        def group(g, carry):
            chunk(2 * g, 0)
            chunk(2 * g + 1, 1)
            return carry

        for s in range(nstages):
            pltpu.sync_copy(edges_hbm.at[0, wid, pl.ds(s * spc, spc)], src_v)
            pltpu.sync_copy(edges_hbm.at[1, wid, pl.ds(s * spc, spc)], dst_v)
            pltpu.async_copy(x_hbm.at[src_v.at[0]], rows_v.at[0], sem0)
            lax.fori_loop(0, spc // 2, group, 0)
        plsc.subcore_barrier()

        pltpu.sync_copy(acc_sh.at[pl.ds(base, RPT)],
                        out_hbm.at[core, pl.ds(base, RPT)])
        if with_cnt:
            pltpu.sync_copy(cnt_sh.at[pl.ds(base, RPT)],
                            cnt_hbm.at[core, pl.ds(base, RPT)])

    return sc_segsum


CH0, CPW0, NST0 = 125, 80, 5
CH1, CPW1, NST1 = 125, 80, 2
_sc_segsum_cnt = _make_sc_segsum(True, CH0, CPW0, NST0)
_sc_segsum = _make_sc_segsum(False, CH1, CPW1, NST1)


_R = 1000


def _dot_t(a, w):
    return jax.lax.dot_general(
        a, w, (((1,), (1,)), ((), ())),
        preferred_element_type=jnp.float32)


def _mean_rows(p_ref, cnt_ref):
    i = pl.program_id(0)
    s = p_ref[0] + p_ref[1]
    c = (cnt_ref[0, pl.ds(i * _R, _R), 0:1]
         + cnt_ref[1, pl.ds(i * _R, _R), 0:1])
    return s / jnp.maximum(c, 1.0)


def _dense_ln_body(p_ref, cnt_ref, x_ref, wl_ref, wr_ref, bl_ref, g_ref,
                   b_ref, o_ref):
    mean = _mean_rows(p_ref, cnt_ref)
    x = x_ref[...]
    h = _dot_t(mean, wl_ref[...]) + _dot_t(x, wr_ref[...]) + bl_ref[...]
    m = jnp.mean(h, axis=-1, keepdims=True)
    v = jnp.mean((h - m) * (h - m), axis=-1, keepdims=True)
    hn = (h - m) / jnp.sqrt(v + 1e-5) * g_ref[...] + b_ref[...]
    o_ref[...] = jnp.maximum(hn, 0.0) + x


def _dense_plain_body(p_ref, cnt_ref, x_ref, wl_ref, wr_ref, bl_ref, o_ref):
    mean = _mean_rows(p_ref, cnt_ref)
    h = _dot_t(mean, wl_ref[...]) + _dot_t(x_ref[...], wr_ref[...])
    o_ref[...] = h + bl_ref[...]


def _dense(body, n_extra):
    in_specs = [
        pl.BlockSpec((NC, _R, D), lambda i: (0, i, 0)),
        pl.BlockSpec((NC, NP, 16), lambda i: (0, 0, 0)),
        pl.BlockSpec((_R, D), lambda i: (i, 0)),
        pl.BlockSpec((D, D), lambda i: (0, 0)),
        pl.BlockSpec((D, D), lambda i: (0, 0)),
        pl.BlockSpec((1, D), lambda i: (0, 0)),
    ]
    in_specs += [pl.BlockSpec((1, D), lambda i: (0, 0))] * n_extra
    return pl.pallas_call(
        body,
        grid=(N // _R,),
        in_specs=in_specs,
        out_specs=pl.BlockSpec((_R, D), lambda i: (i, 0)),
        out_shape=jax.ShapeDtypeStruct((N, D), jnp.float32),
    )


_dense_ln = _dense(_dense_ln_body, 2)
_dense_plain = _dense(_dense_plain_body, 0)


def kernel(x, edge_index, Wl0, bl0, Wr0, Wl1, bl1, Wr1, Wl2, bl2, Wr2,
           g0, b0, g1, b1):
    edges = edge_index.reshape(2, NW, CPW0, CH0)

    def r(a):
        return a.reshape(1, D)

    p, cnt = _sc_segsum_cnt(x, edges)
    x1 = _dense_ln(p, cnt, x, Wl0, Wr0, r(bl0), r(g0), r(b0))
    (p,) = _sc_segsum(x1, edges)
    x2 = _dense_ln(p, cnt, x1, Wl1, Wr1, r(bl1), r(g1), r(b1))
    (p,) = _sc_segsum(x2, edges)
    return _dense_plain(p, cnt, x2, Wl2, Wr2, r(bl2))
